# baseline (device time: 108948 ns/iter reference)
import math

import jax
import jax.numpy as jnp
from jax import lax
from jax.experimental import pallas as pl
from jax.experimental.pallas import tpu as pltpu

Q_CHUNK = 512
HEADS_PER_FLOW = 4


def kernel(Q, K, V):
    b, s, h, d = Q.shape
    n_chunks = s // Q_CHUNK
    q_const = (d ** -0.5) * math.log2(math.e)

    def body(q_hbm, k_hbm, v_hbm, o_hbm,
             kst, vst, qst, kb, vb, qb, ck, cv, ov, l_scr,
             kd_sem, vd_sem, qd_sem, od_sem,
             send_k, send_v, recv_k, recv_v):
        my_x = lax.axis_index("x")
        my_y = lax.axis_index("y")
        my_z = lax.axis_index("z")
        peer = (my_x, 1 - my_y, my_z)

        def in_dma(hbm, stage, sem, hh):
            return pltpu.make_async_copy(
                hbm.at[0, :, hh, :], stage.at[hh], sem.at[hh]
            )

        def rdma(ff, which):
            src, dst = (kb, ck) if which == 0 else (vb, cv)
            ss, rs = (send_k, recv_k) if which == 0 else (send_v, recv_v)
            lo = ff * HEADS_PER_FLOW
            return pltpu.make_async_remote_copy(
                src_ref=src.at[pl.ds(lo, HEADS_PER_FLOW)],
                dst_ref=dst.at[pl.ds(lo, HEADS_PER_FLOW)],
                send_sem=ss.at[ff], recv_sem=rs.at[ff],
                device_id=peer, device_id_type=pl.DeviceIdType.MESH,
            )

        barrier_sem = pltpu.get_barrier_semaphore()
        pl.semaphore_signal(
            barrier_sem, inc=1, device_id=peer,
            device_id_type=pl.DeviceIdType.MESH,
        )
        pl.semaphore_wait(barrier_sem, 1)

        for hh in range(h):
            in_dma(k_hbm, kst, kd_sem, hh).start()
            in_dma(v_hbm, vst, vd_sem, hh).start()
        for ff in range(h // HEADS_PER_FLOW):
            for hh in range(ff * HEADS_PER_FLOW, (ff + 1) * HEADS_PER_FLOW):
                in_dma(k_hbm, kst, kd_sem, hh).wait()
                kb[hh] = kst[hh].astype(jnp.bfloat16)
            rdma(ff, 0).start()
            for hh in range(ff * HEADS_PER_FLOW, (ff + 1) * HEADS_PER_FLOW):
                in_dma(v_hbm, vst, vd_sem, hh).wait()
                vb[hh] = vst[hh].astype(jnp.bfloat16)
            rdma(ff, 1).start()

        for hh in range(h):
            in_dma(q_hbm, qst, qd_sem, hh).start()
        for hh in range(h):
            in_dma(q_hbm, qst, qd_sem, hh).wait()
            qb[hh] = (qst[hh] * q_const).astype(jnp.bfloat16)

        def phase1(i, _):
            head = i // n_chunks
            qc = i % n_chunks
            q = qb[head, pl.ds(qc * Q_CHUNK, Q_CHUNK), :]
            s1 = lax.dot_general(
                q, kb[head], (((1,), (1,)), ((), ())),
                preferred_element_type=jnp.float32,
            )
            p1 = jnp.exp2(s1)
            l1 = jnp.sum(p1, axis=1, keepdims=True)
            o1 = lax.dot_general(
                p1.astype(jnp.bfloat16), vb[head], (((1,), (0,)), ((), ())),
                preferred_element_type=jnp.float32,
            )
            ov[head, pl.ds(qc * Q_CHUNK, Q_CHUNK), :] = o1
            l_scr[i, :] = l1[:, 0]
            return 0

        lax.fori_loop(0, h * n_chunks, phase1, 0)

        for head in range(h):
            if head % HEADS_PER_FLOW == 0:
                rdma(head // HEADS_PER_FLOW, 0).wait_recv()
                rdma(head // HEADS_PER_FLOW, 1).wait_recv()
            for qc in range(n_chunks):
                i = head * n_chunks + qc
                q = qb[head, pl.ds(qc * Q_CHUNK, Q_CHUNK), :]
                s2 = lax.dot_general(
                    q, ck[head], (((1,), (1,)), ((), ())),
                    preferred_element_type=jnp.float32,
                )
                p2 = jnp.exp2(s2)
                l2 = jnp.sum(p2, axis=1, keepdims=True)
                o2 = lax.dot_general(
                    p2.astype(jnp.bfloat16), cv[head], (((1,), (0,)), ((), ())),
                    preferred_element_type=jnp.float32,
                )
                l1 = l_scr[i, :][:, None]
                o1 = ov[head, pl.ds(qc * Q_CHUNK, Q_CHUNK), :]
                ov[head, pl.ds(qc * Q_CHUNK, Q_CHUNK), :] = (
                    (o1 + o2) / (l1 + l2)
                )
            pltpu.make_async_copy(
                ov.at[head], o_hbm.at[0, :, head, :], od_sem.at[head]
            ).start()

        for head in range(h):
            pltpu.make_async_copy(
                ov.at[head], o_hbm.at[0, :, head, :], od_sem.at[head]
            ).wait()
        for ff in range(h // HEADS_PER_FLOW):
            rdma(ff, 0).wait_send()
            rdma(ff, 1).wait_send()

    out = pl.pallas_call(
        body,
        out_shape=jax.ShapeDtypeStruct((b, s, h, d), jnp.float32),
        in_specs=[pl.BlockSpec(memory_space=pl.ANY)] * 3,
        out_specs=pl.BlockSpec(memory_space=pl.ANY),
        scratch_shapes=[
            pltpu.VMEM((h, s, d), jnp.float32),
            pltpu.VMEM((h, s, d), jnp.float32),
            pltpu.VMEM((h, s, d), jnp.float32),
            pltpu.VMEM((h, s, d), jnp.bfloat16),
            pltpu.VMEM((h, s, d), jnp.bfloat16),
            pltpu.VMEM((h, s, d), jnp.bfloat16),
            pltpu.VMEM((h, s, d), jnp.bfloat16),
            pltpu.VMEM((h, s, d), jnp.bfloat16),
            pltpu.VMEM((h, s, d), jnp.float32),
            pltpu.VMEM((h * (s // Q_CHUNK), Q_CHUNK), jnp.float32),
            pltpu.SemaphoreType.DMA((16,)),
            pltpu.SemaphoreType.DMA((16,)),
            pltpu.SemaphoreType.DMA((16,)),
            pltpu.SemaphoreType.DMA((16,)),
            pltpu.SemaphoreType.DMA((16,)),
            pltpu.SemaphoreType.DMA((16,)),
            pltpu.SemaphoreType.DMA((16,)),
            pltpu.SemaphoreType.DMA((16,)),
        ],
        compiler_params=pltpu.CompilerParams(
            collective_id=0,
            vmem_limit_bytes=63 * 1024 * 1024,
        ),
    )(Q, K, V)

    return out


# device time: 104244 ns/iter; 1.0451x vs baseline; 1.0451x over previous
import math
import os

import jax
import jax.numpy as jnp
from jax import lax
from jax.experimental import pallas as pl
from jax.experimental.pallas import tpu as pltpu

Q_CHUNK = 512
HEADS_PER_FLOW = 1
_VARIANT = os.environ.get("KVAR", "full")
_DO_COMM = _VARIANT in ("comm", "full")
_DO_COMPUTE = _VARIANT in ("compute", "full")


def kernel(Q, K, V):
    b, s, h, d = Q.shape
    n_chunks = s // Q_CHUNK
    q_const = (d ** -0.5) * math.log2(math.e)

    def body(q_hbm, k_hbm, v_hbm, o_hbm,
             kst, vst, qst, kb, vb, qb, ck, cv, ov, l_scr,
             kd_sem, vd_sem, qd_sem, od_sem,
             send_k, send_v, recv_k, recv_v):
        my_x = lax.axis_index("x")
        my_y = lax.axis_index("y")
        my_z = lax.axis_index("z")
        peer = (my_x, 1 - my_y, my_z)

        def in_dma(hbm, stage, sem, hh):
            return pltpu.make_async_copy(
                hbm.at[0, :, hh, :], stage.at[hh], sem.at[hh]
            )

        def rdma(ff, which):
            src, dst = (kb, ck) if which == 0 else (vb, cv)
            ss, rs = (send_k, recv_k) if which == 0 else (send_v, recv_v)
            lo = ff * HEADS_PER_FLOW
            return pltpu.make_async_remote_copy(
                src_ref=src.at[pl.ds(lo, HEADS_PER_FLOW)],
                dst_ref=dst.at[pl.ds(lo, HEADS_PER_FLOW)],
                send_sem=ss.at[ff], recv_sem=rs.at[ff],
                device_id=peer, device_id_type=pl.DeviceIdType.MESH,
            )

        barrier_sem = pltpu.get_barrier_semaphore()
        pl.semaphore_signal(
            barrier_sem, inc=1, device_id=peer,
            device_id_type=pl.DeviceIdType.MESH,
        )
        pl.semaphore_wait(barrier_sem, 1)

        for hh in range(h):
            in_dma(k_hbm, kst, kd_sem, hh).start()
            in_dma(v_hbm, vst, vd_sem, hh).start()
        for ff in range(h // HEADS_PER_FLOW):
            for hh in range(ff * HEADS_PER_FLOW, (ff + 1) * HEADS_PER_FLOW):
                in_dma(k_hbm, kst, kd_sem, hh).wait()
                kb[hh] = kst[hh].astype(jnp.bfloat16)
            if _DO_COMM:
                rdma(ff, 0).start()
            for hh in range(ff * HEADS_PER_FLOW, (ff + 1) * HEADS_PER_FLOW):
                in_dma(v_hbm, vst, vd_sem, hh).wait()
                vb[hh] = vst[hh].astype(jnp.bfloat16)
            if _DO_COMM:
                rdma(ff, 1).start()
        if not _DO_COMM:
            ck[...] = kb[...]
            cv[...] = vb[...]

        for hh in range(h):
            in_dma(q_hbm, qst, qd_sem, hh).start()
        for hh in range(h):
            in_dma(q_hbm, qst, qd_sem, hh).wait()
            qb[hh] = (qst[hh] * q_const).astype(jnp.bfloat16)

        def phase1(i, _):
            head = i // n_chunks
            qc = i % n_chunks
            q = qb[head, pl.ds(qc * Q_CHUNK, Q_CHUNK), :]
            s1 = lax.dot_general(
                q, kb[head], (((1,), (1,)), ((), ())),
                preferred_element_type=jnp.float32,
            )
            p1 = jnp.exp2(s1)
            l1 = jnp.sum(p1, axis=1, keepdims=True)
            o1 = lax.dot_general(
                p1.astype(jnp.bfloat16), vb[head], (((1,), (0,)), ((), ())),
                preferred_element_type=jnp.float32,
            )
            ov[head, pl.ds(qc * Q_CHUNK, Q_CHUNK), :] = o1
            l_scr[i, :] = l1[:, 0]
            return 0

        if _DO_COMPUTE:
            lax.fori_loop(0, h * n_chunks, phase1, 0)

        for head in range(h):
            if _DO_COMM and head % HEADS_PER_FLOW == 0:
                rdma(head // HEADS_PER_FLOW, 0).wait_recv()
                rdma(head // HEADS_PER_FLOW, 1).wait_recv()
            if not _DO_COMPUTE:
                continue
            for qc in range(n_chunks):
                i = head * n_chunks + qc
                q = qb[head, pl.ds(qc * Q_CHUNK, Q_CHUNK), :]
                s2 = lax.dot_general(
                    q, ck[head], (((1,), (1,)), ((), ())),
                    preferred_element_type=jnp.float32,
                )
                p2 = jnp.exp2(s2)
                l2 = jnp.sum(p2, axis=1, keepdims=True)
                o2 = lax.dot_general(
                    p2.astype(jnp.bfloat16), cv[head], (((1,), (0,)), ((), ())),
                    preferred_element_type=jnp.float32,
                )
                l1 = l_scr[i, :][:, None]
                o1 = ov[head, pl.ds(qc * Q_CHUNK, Q_CHUNK), :]
                ov[head, pl.ds(qc * Q_CHUNK, Q_CHUNK), :] = (
                    (o1 + o2) / (l1 + l2)
                )
            pltpu.make_async_copy(
                ov.at[head], o_hbm.at[0, :, head, :], od_sem.at[head]
            ).start()

        if _DO_COMPUTE:
            for head in range(h):
                pltpu.make_async_copy(
                    ov.at[head], o_hbm.at[0, :, head, :], od_sem.at[head]
                ).wait()
        if _DO_COMM:
            for ff in range(h // HEADS_PER_FLOW):
                rdma(ff, 0).wait_send()
                rdma(ff, 1).wait_send()

    out = pl.pallas_call(
        body,
        out_shape=jax.ShapeDtypeStruct((b, s, h, d), jnp.float32),
        in_specs=[pl.BlockSpec(memory_space=pl.ANY)] * 3,
        out_specs=pl.BlockSpec(memory_space=pl.ANY),
        scratch_shapes=[
            pltpu.VMEM((h, s, d), jnp.float32),
            pltpu.VMEM((h, s, d), jnp.float32),
            pltpu.VMEM((h, s, d), jnp.float32),
            pltpu.VMEM((h, s, d), jnp.bfloat16),
            pltpu.VMEM((h, s, d), jnp.bfloat16),
            pltpu.VMEM((h, s, d), jnp.bfloat16),
            pltpu.VMEM((h, s, d), jnp.bfloat16),
            pltpu.VMEM((h, s, d), jnp.bfloat16),
            pltpu.VMEM((h, s, d), jnp.float32),
            pltpu.VMEM((h * (s // Q_CHUNK), Q_CHUNK), jnp.float32),
            pltpu.SemaphoreType.DMA((16,)),
            pltpu.SemaphoreType.DMA((16,)),
            pltpu.SemaphoreType.DMA((16,)),
            pltpu.SemaphoreType.DMA((16,)),
            pltpu.SemaphoreType.DMA((16,)),
            pltpu.SemaphoreType.DMA((16,)),
            pltpu.SemaphoreType.DMA((16,)),
            pltpu.SemaphoreType.DMA((16,)),
        ],
        compiler_params=pltpu.CompilerParams(
            collective_id=0,
            vmem_limit_bytes=63 * 1024 * 1024,
        ),
    )(Q, K, V)

    return out


# device time: 104229 ns/iter; 1.0453x vs baseline; 1.0001x over previous
import math
import os

import jax
import jax.numpy as jnp
from jax import lax
from jax.experimental import pallas as pl
from jax.experimental.pallas import tpu as pltpu

Q_CHUNK = 512
HEADS_PER_FLOW = 1
_VARIANT = os.environ.get("KVAR", "full")
_DO_COMM = _VARIANT in ("comm", "comm_pure", "full")
_DO_STAGE = _VARIANT != "comm_pure"
_DO_COMPUTE = _VARIANT in ("compute", "full")


def kernel(Q, K, V):
    b, s, h, d = Q.shape
    n_chunks = s // Q_CHUNK
    q_const = (d ** -0.5) * math.log2(math.e)

    def body(q_hbm, k_hbm, v_hbm, o_hbm,
             kst, vst, qst, kb, vb, qb, ck, cv, ov, l_scr,
             kd_sem, vd_sem, qd_sem, od_sem,
             send_k, send_v, recv_k, recv_v):
        my_x = lax.axis_index("x")
        my_y = lax.axis_index("y")
        my_z = lax.axis_index("z")
        peer = (my_x, 1 - my_y, my_z)

        def in_dma(hbm, stage, sem, hh):
            return pltpu.make_async_copy(
                hbm.at[0, :, hh, :], stage.at[hh], sem.at[hh]
            )

        def rdma(ff, which):
            src, dst = (kb, ck) if which == 0 else (vb, cv)
            ss, rs = (send_k, recv_k) if which == 0 else (send_v, recv_v)
            lo = ff * HEADS_PER_FLOW
            return pltpu.make_async_remote_copy(
                src_ref=src.at[pl.ds(lo, HEADS_PER_FLOW)],
                dst_ref=dst.at[pl.ds(lo, HEADS_PER_FLOW)],
                send_sem=ss.at[ff], recv_sem=rs.at[ff],
                device_id=peer, device_id_type=pl.DeviceIdType.MESH,
            )

        barrier_sem = pltpu.get_barrier_semaphore()
        pl.semaphore_signal(
            barrier_sem, inc=1, device_id=peer,
            device_id_type=pl.DeviceIdType.MESH,
        )
        pl.semaphore_wait(barrier_sem, 1)

        if _DO_STAGE:
            for hh in range(h):
                in_dma(k_hbm, kst, kd_sem, hh).start()
                in_dma(v_hbm, vst, vd_sem, hh).start()
        for ff in range(h // HEADS_PER_FLOW):
            for hh in range(ff * HEADS_PER_FLOW, (ff + 1) * HEADS_PER_FLOW):
                if _DO_STAGE:
                    in_dma(k_hbm, kst, kd_sem, hh).wait()
                    kb[hh] = kst[hh].astype(jnp.bfloat16)
            if _DO_COMM:
                rdma(ff, 0).start()
            for hh in range(ff * HEADS_PER_FLOW, (ff + 1) * HEADS_PER_FLOW):
                if _DO_STAGE:
                    in_dma(v_hbm, vst, vd_sem, hh).wait()
                    vb[hh] = vst[hh].astype(jnp.bfloat16)
            if _DO_COMM:
                rdma(ff, 1).start()
        if not _DO_COMM:
            ck[...] = kb[...]
            cv[...] = vb[...]

        if _DO_STAGE:
            for hh in range(h):
                in_dma(q_hbm, qst, qd_sem, hh).start()
            for hh in range(h):
                in_dma(q_hbm, qst, qd_sem, hh).wait()
                qb[hh] = (qst[hh] * q_const).astype(jnp.bfloat16)

        def phase1(i, _):
            head = i // n_chunks
            qc = i % n_chunks
            q = qb[head, pl.ds(qc * Q_CHUNK, Q_CHUNK), :]
            s1 = lax.dot_general(
                q, kb[head], (((1,), (1,)), ((), ())),
                preferred_element_type=jnp.float32,
            )
            p1 = jnp.exp2(s1)
            l1 = jnp.sum(p1, axis=1, keepdims=True)
            o1 = lax.dot_general(
                p1.astype(jnp.bfloat16), vb[head], (((1,), (0,)), ((), ())),
                preferred_element_type=jnp.float32,
            )
            ov[head, pl.ds(qc * Q_CHUNK, Q_CHUNK), :] = o1
            l_scr[i, :] = l1[:, 0]
            return 0

        if _DO_COMPUTE:
            lax.fori_loop(0, h * n_chunks, phase1, 0)

        for head in range(h):
            if _DO_COMM and head % HEADS_PER_FLOW == 0:
                rdma(head // HEADS_PER_FLOW, 0).wait_recv()
                rdma(head // HEADS_PER_FLOW, 1).wait_recv()
            if not _DO_COMPUTE:
                continue
            for qc in range(n_chunks):
                i = head * n_chunks + qc
                q = qb[head, pl.ds(qc * Q_CHUNK, Q_CHUNK), :]
                s2 = lax.dot_general(
                    q, ck[head], (((1,), (1,)), ((), ())),
                    preferred_element_type=jnp.float32,
                )
                p2 = jnp.exp2(s2)
                l2 = jnp.sum(p2, axis=1, keepdims=True)
                o2 = lax.dot_general(
                    p2.astype(jnp.bfloat16), cv[head], (((1,), (0,)), ((), ())),
                    preferred_element_type=jnp.float32,
                )
                l1 = l_scr[i, :][:, None]
                o1 = ov[head, pl.ds(qc * Q_CHUNK, Q_CHUNK), :]
                ov[head, pl.ds(qc * Q_CHUNK, Q_CHUNK), :] = (
                    (o1 + o2) / (l1 + l2)
                )
            pltpu.make_async_copy(
                ov.at[head], o_hbm.at[0, :, head, :], od_sem.at[head]
            ).start()

        if _DO_COMPUTE:
            for head in range(h):
                pltpu.make_async_copy(
                    ov.at[head], o_hbm.at[0, :, head, :], od_sem.at[head]
                ).wait()
        if _DO_COMM:
            for ff in range(h // HEADS_PER_FLOW):
                rdma(ff, 0).wait_send()
                rdma(ff, 1).wait_send()

    out = pl.pallas_call(
        body,
        out_shape=jax.ShapeDtypeStruct((b, s, h, d), jnp.float32),
        in_specs=[pl.BlockSpec(memory_space=pl.ANY)] * 3,
        out_specs=pl.BlockSpec(memory_space=pl.ANY),
        scratch_shapes=[
            pltpu.VMEM((h, s, d), jnp.float32),
            pltpu.VMEM((h, s, d), jnp.float32),
            pltpu.VMEM((h, s, d), jnp.float32),
            pltpu.VMEM((h, s, d), jnp.bfloat16),
            pltpu.VMEM((h, s, d), jnp.bfloat16),
            pltpu.VMEM((h, s, d), jnp.bfloat16),
            pltpu.VMEM((h, s, d), jnp.bfloat16),
            pltpu.VMEM((h, s, d), jnp.bfloat16),
            pltpu.VMEM((h, s, d), jnp.float32),
            pltpu.VMEM((h * (s // Q_CHUNK), Q_CHUNK), jnp.float32),
            pltpu.SemaphoreType.DMA((16,)),
            pltpu.SemaphoreType.DMA((16,)),
            pltpu.SemaphoreType.DMA((16,)),
            pltpu.SemaphoreType.DMA((16,)),
            pltpu.SemaphoreType.DMA((16,)),
            pltpu.SemaphoreType.DMA((16,)),
            pltpu.SemaphoreType.DMA((16,)),
            pltpu.SemaphoreType.DMA((16,)),
        ],
        compiler_params=pltpu.CompilerParams(
            collective_id=0,
            vmem_limit_bytes=63 * 1024 * 1024,
        ),
    )(Q, K, V)

    return out
